# baseline (speedup 1.0000x reference)
import functools

import jax
import jax.numpy as jnp
from jax import lax
from jax.experimental import pallas as pl
from jax.experimental.pallas import tpu as pltpu
from jax.experimental.pallas import tpu_sc as plsc

NC = 2
NS = 16
NW = NC * NS
LANES = 16
CP1 = 128
CP2 = 256


def _mesh():
    return plsc.VectorSubcoreMesh(
        core_axis_name="c", subcore_axis_name="s", num_cores=NC, num_subcores=NS
    )


_CPARAMS = pltpu.CompilerParams(needs_layout_passes=False)


def _wid():
    return lax.axis_index("s") * NC + lax.axis_index("c")


def _chunk_range(wid, nch):
    lo = nch // NW
    hi = nch % NW
    start = wid * lo + jnp.minimum(wid, hi)
    cnt = lo + jnp.where(wid < hi, 1, 0)
    return start, cnt


def _make_stage1(P, L, D, T):
    NCH = P // CP1
    PTAIL = P % CP1
    TB = NCH * CP1
    assert PTAIL % LANES == 0

    @functools.partial(
        pl.kernel,
        out_type=jax.ShapeDtypeStruct((P // 2,), jnp.int32),
        mesh=_mesh(),
        compiler_params=_CPARAMS,
        scratch_types=[
            pltpu.VMEM((T,), jnp.float32),
            pltpu.VMEM((L, D, CP1), jnp.int32),
            pltpu.VMEM((L, D, CP1), jnp.int32),
            pltpu.VMEM((L, CP1), jnp.int32),
            pltpu.VMEM((L, CP1), jnp.int32),
            pltpu.VMEM((CP1 // 2,), jnp.int32),
            pltpu.VMEM((CP1 // 2,), jnp.int32),
            pltpu.VMEM((CP1,), jnp.float32),
            pltpu.SemaphoreType.DMA,
            pltpu.SemaphoreType.DMA,
            pltpu.SemaphoreType.DMA,
            pltpu.SemaphoreType.DMA,
        ],
    )
    def k(nom_t, den_t, w_hbm, out_hbm,
          table_v, den0, den1, nom0, nom1, out0, out1, stage_v, s0, s1, so0, so1):
        wid = _wid()
        start_ch, cnt_ch = _chunk_range(wid, NCH)
        npairs = (cnt_ch + 1) // 2

        def cbase(ci):
            return (start_ch + jnp.minimum(ci, cnt_ch - 1)) * CP1

        def cbase2(ci):
            return (start_ch + jnp.minimum(ci, cnt_ch - 1)) * (CP1 // 2)

        def issue(ci, den_b, nom_b, sem):
            b = cbase(ci)
            pltpu.async_copy(den_t.at[:, :, pl.ds(b, CP1)], den_b, sem)
            pltpu.async_copy(nom_t.at[:, pl.ds(b, CP1)], nom_b, sem)

        def wait_in(den_b, nom_b, sem):
            pltpu.make_async_copy(den_t.at[:, :, pl.ds(0, CP1)], den_b, sem).wait()
            pltpu.make_async_copy(nom_t.at[:, pl.ds(0, CP1)], nom_b, sem).wait()

        def wait_out(out_b, sem):
            pltpu.make_async_copy(out_b, out_hbm.at[pl.ds(0, CP1 // 2)], sem).wait()

        def block(den_b, nom_b, j):
            sl = pl.ds(j * LANES, LANES)
            prob = None
            for l in range(L):
                nw = jnp.abs(plsc.load_gather(table_v, [nom_b[l, sl]]))
                dsum = jnp.abs(plsc.load_gather(table_v, [den_b[l, 0, sl]]))
                for d in range(1, D):
                    dsum = dsum + jnp.abs(
                        plsc.load_gather(table_v, [den_b[l, d, sl]])
                    )
                f = nw / dsum
                prob = f if prob is None else prob * f
            stage_v[sl] = prob

        def pack2(out_b, j2):
            pa = stage_v[pl.ds(2 * j2 * LANES, LANES)]
            pb = stage_v[pl.ds((2 * j2 + 1) * LANES, LANES)]
            words = plsc.bitcast(
                plsc.pack(pa, pb, format=plsc.PackFormat.INTERLEAVED), jnp.int32
            )
            out_b[pl.ds(j2 * LANES, LANES)] = words

        def compute(den_b, nom_b, out_b):
            @plsc.parallel_loop(0, CP1 // LANES, step=1, unroll=3)
            def _(j):
                block(den_b, nom_b, j)

            @plsc.parallel_loop(0, CP1 // (2 * LANES), step=1, unroll=1)
            def _(j2):
                pack2(out_b, j2)

        pltpu.sync_copy(w_hbm, table_v)
        issue(0, den0, nom0, s0)

        def body(i, carry):
            ci = 2 * i
            issue(ci + 1, den1, nom1, s1)
            wait_in(den0, nom0, s0)

            @pl.when(i > 0)
            def _():
                wait_out(out0, so0)

            compute(den0, nom0, out0)
            pltpu.async_copy(out0, out_hbm.at[pl.ds(cbase2(ci), CP1 // 2)], so0)

            wait_in(den1, nom1, s1)
            issue(ci + 2, den0, nom0, s0)

            @pl.when(i > 0)
            def _():
                wait_out(out1, so1)

            compute(den1, nom1, out1)
            pltpu.async_copy(out1, out_hbm.at[pl.ds(cbase2(ci + 1), CP1 // 2)], so1)
            return carry

        lax.fori_loop(0, npairs, body, 0)
        wait_in(den0, nom0, s0)
        wait_out(out0, so0)
        wait_out(out1, so1)

        if PTAIL:
            @pl.when(wid == NW - 1)
            def _():
                for l in range(L):
                    for d in range(D):
                        pltpu.async_copy(
                            den_t.at[l, d, pl.ds(TB, PTAIL)],
                            den1.at[l, d, pl.ds(0, PTAIL)], s1,
                        )
                    pltpu.async_copy(
                        nom_t.at[l, pl.ds(TB, PTAIL)],
                        nom1.at[l, pl.ds(0, PTAIL)], s1,
                    )
                for l in range(L):
                    for d in range(D):
                        pltpu.make_async_copy(
                            den_t.at[l, d, pl.ds(TB, PTAIL)],
                            den1.at[l, d, pl.ds(0, PTAIL)], s1,
                        ).wait()
                    pltpu.make_async_copy(
                        nom_t.at[l, pl.ds(TB, PTAIL)],
                        nom1.at[l, pl.ds(0, PTAIL)], s1,
                    ).wait()
                for j in range(PTAIL // LANES):
                    block(den1, nom1, j)
                for j2 in range(PTAIL // (2 * LANES)):
                    pack2(out1, j2)
                pltpu.sync_copy(
                    out1.at[pl.ds(0, PTAIL // 2)],
                    out_hbm.at[pl.ds(TB // 2, PTAIL // 2)],
                )

    return k


def _make_stage2(V, K, P):
    NCH = V // CP2
    PTAIL = V % CP2
    TB = NCH * CP2
    PW = P // 2
    assert PTAIL % LANES == 0 and P % 32 == 0

    @functools.partial(
        pl.kernel,
        out_type=jax.ShapeDtypeStruct((V,), jnp.float32),
        mesh=_mesh(),
        compiler_params=_CPARAMS,
        scratch_types=[
            pltpu.VMEM((PW,), jnp.int32),
            pltpu.VMEM((K, CP2), jnp.int32),
            pltpu.VMEM((K, CP2), jnp.int32),
            pltpu.VMEM((CP2,), jnp.float32),
            pltpu.VMEM((CP2,), jnp.float32),
            pltpu.SemaphoreType.DMA,
            pltpu.SemaphoreType.DMA,
            pltpu.SemaphoreType.DMA,
            pltpu.SemaphoreType.DMA,
        ],
    )
    def k(v2p_t, pairs_hbm, out_hbm,
          pairs_v, vb0, vb1, out0, out1, s0, s1, so0, so1):
        wid = _wid()
        start_ch, cnt_ch = _chunk_range(wid, NCH)
        npairs = (cnt_ch + 1) // 2

        def cbase(ci):
            return (start_ch + jnp.minimum(ci, cnt_ch - 1)) * CP2

        def issue(ci, vb, sem):
            pltpu.async_copy(v2p_t.at[:, pl.ds(cbase(ci), CP2)], vb, sem)

        def wait_in(vb, sem):
            pltpu.make_async_copy(v2p_t.at[:, pl.ds(0, CP2)], vb, sem).wait()

        def wait_out(out_b, sem):
            pltpu.make_async_copy(out_b, out_hbm.at[pl.ds(0, CP2)], sem).wait()

        def block(vb, out_b, j):
            sl = pl.ds(j * LANES, LANES)
            acc = jnp.zeros((LANES,), jnp.float32)
            for kk in range(K):
                idx = vb[kk, sl]
                widx = ((idx >> 5) << 4) | (idx & 15)
                g = plsc.load_gather(pairs_v, [widx])
                hi = (idx & 16) != 0
                bits = jnp.where(hi, g & jnp.int32(-65536), g << 16)
                acc = acc + plsc.bitcast(bits, jnp.float32)
            out_b[sl] = acc

        pltpu.sync_copy(pairs_hbm, pairs_v)
        issue(0, vb0, s0)

        def body(i, carry):
            ci = 2 * i
            issue(ci + 1, vb1, s1)
            wait_in(vb0, s0)

            @pl.when(i > 0)
            def _():
                wait_out(out0, so0)

            @plsc.parallel_loop(0, CP2 // LANES, step=1, unroll=2)
            def _(j):
                block(vb0, out0, j)

            pltpu.async_copy(out0, out_hbm.at[pl.ds(cbase(ci), CP2)], so0)

            wait_in(vb1, s1)
            issue(ci + 2, vb0, s0)

            @pl.when(i > 0)
            def _():
                wait_out(out1, so1)

            @plsc.parallel_loop(0, CP2 // LANES, step=1, unroll=2)
            def _(j):
                block(vb1, out1, j)

            pltpu.async_copy(out1, out_hbm.at[pl.ds(cbase(ci + 1), CP2)], so1)
            return carry

        lax.fori_loop(0, npairs, body, 0)
        wait_in(vb0, s0)
        wait_out(out0, so0)
        wait_out(out1, so1)

        if PTAIL:
            @pl.when(wid == NW - 1)
            def _():
                for kk in range(K):
                    pltpu.async_copy(
                        v2p_t.at[kk, pl.ds(TB, PTAIL)],
                        vb1.at[kk, pl.ds(0, PTAIL)], s1,
                    )
                for kk in range(K):
                    pltpu.make_async_copy(
                        v2p_t.at[kk, pl.ds(TB, PTAIL)],
                        vb1.at[kk, pl.ds(0, PTAIL)], s1,
                    ).wait()
                for j in range(PTAIL // LANES):
                    block(vb1, out1, j)
                pltpu.sync_copy(
                    out1.at[pl.ds(0, PTAIL)], out_hbm.at[pl.ds(TB, PTAIL)]
                )

    return k


def kernel(variant_2_paths, paths_nom, paths_denom, w_transitions):
    V, K = variant_2_paths.shape
    P, L = paths_nom.shape
    D = paths_denom.shape[-1]
    T = w_transitions.shape[0]

    v2p_t = variant_2_paths.astype(jnp.int32).T
    nom_t = paths_nom.astype(jnp.int32).T
    den_t = jnp.transpose(paths_denom.astype(jnp.int32), (1, 2, 0))
    w = w_transitions.astype(jnp.float32)

    prob = _make_stage1(P, L, D, T)(nom_t, den_t, w)
    return _make_stage2(V, K, P)(v2p_t, prob)


# speedup vs baseline: 1.2649x; 1.2649x over previous
import functools

import jax
import jax.numpy as jnp
from jax import lax
from jax.experimental import pallas as pl
from jax.experimental.pallas import tpu as pltpu
from jax.experimental.pallas import tpu_sc as plsc

NC = 2
NS = 16
NW = NC * NS
LANES = 16
CP1 = 128
CP2 = 256


def _mesh():
    return plsc.VectorSubcoreMesh(
        core_axis_name="c", subcore_axis_name="s", num_cores=NC, num_subcores=NS
    )


_CPARAMS = pltpu.CompilerParams(needs_layout_passes=False)


def _wid():
    return lax.axis_index("s") * NC + lax.axis_index("c")


def _chunk_range(wid, nch):
    lo = nch // NW
    hi = nch % NW
    start = wid * lo + jnp.minimum(wid, hi)
    cnt = lo + jnp.where(wid < hi, 1, 0)
    return start, cnt


def _make_stage1(P, L, D, T):
    NCH = P // CP1
    PTAIL = P % CP1
    TB = NCH * CP1
    assert PTAIL % LANES == 0

    @functools.partial(
        pl.kernel,
        out_type=jax.ShapeDtypeStruct((P // 2,), jnp.int32),
        mesh=_mesh(),
        compiler_params=_CPARAMS,
        scratch_types=[
            pltpu.VMEM((T,), jnp.float32),
            pltpu.VMEM((L, D, CP1), jnp.int32),
            pltpu.VMEM((L, D, CP1), jnp.int32),
            pltpu.VMEM((L, CP1), jnp.int32),
            pltpu.VMEM((L, CP1), jnp.int32),
            pltpu.VMEM((CP1 // 2,), jnp.int32),
            pltpu.VMEM((CP1 // 2,), jnp.int32),
            pltpu.VMEM((CP1,), jnp.float32),
            pltpu.SemaphoreType.DMA,
            pltpu.SemaphoreType.DMA,
            pltpu.SemaphoreType.DMA,
            pltpu.SemaphoreType.DMA,
        ],
    )
    def k(nom_t, den_t, w_hbm, out_hbm,
          table_v, den0, den1, nom0, nom1, out0, out1, stage_v, s0, s1, so0, so1):
        wid = _wid()
        start_ch, cnt_ch = _chunk_range(wid, NCH)
        npairs = (cnt_ch + 1) // 2

        def cbase(ci):
            return (start_ch + jnp.minimum(ci, cnt_ch - 1)) * CP1

        def cbase2(ci):
            return (start_ch + jnp.minimum(ci, cnt_ch - 1)) * (CP1 // 2)

        def issue(ci, den_b, nom_b, sem):
            b = cbase(ci)
            pltpu.async_copy(den_t.at[:, :, pl.ds(b, CP1)], den_b, sem)
            pltpu.async_copy(nom_t.at[:, pl.ds(b, CP1)], nom_b, sem)

        def wait_in(den_b, nom_b, sem):
            pltpu.make_async_copy(den_t.at[:, :, pl.ds(0, CP1)], den_b, sem).wait()
            pltpu.make_async_copy(nom_t.at[:, pl.ds(0, CP1)], nom_b, sem).wait()

        def wait_out(out_b, sem):
            pltpu.make_async_copy(out_b, out_hbm.at[pl.ds(0, CP1 // 2)], sem).wait()

        def block(den_b, nom_b, j):
            sl = pl.ds(j * LANES, LANES)
            prob = None
            for l in range(L):
                nw = jnp.abs(plsc.load_gather(table_v, [nom_b[l, sl]]))
                dsum = jnp.abs(plsc.load_gather(table_v, [den_b[l, 0, sl]]))
                for d in range(1, D):
                    dsum = dsum + jnp.abs(
                        plsc.load_gather(table_v, [den_b[l, d, sl]])
                    )
                f = nw / dsum
                prob = f if prob is None else prob * f
            stage_v[sl] = prob

        def pack2(out_b, j2):
            pa = stage_v[pl.ds(2 * j2 * LANES, LANES)]
            pb = stage_v[pl.ds((2 * j2 + 1) * LANES, LANES)]
            words = plsc.bitcast(
                plsc.pack(pa, pb, format=plsc.PackFormat.INTERLEAVED), jnp.int32
            )
            out_b[pl.ds(j2 * LANES, LANES)] = words

        def compute(den_b, nom_b, out_b):
            @plsc.parallel_loop(0, CP1 // LANES, step=1, unroll=2)
            def _(j):
                block(den_b, nom_b, j)

            @plsc.parallel_loop(0, CP1 // (2 * LANES), step=1, unroll=1)
            def _(j2):
                pack2(out_b, j2)

        pltpu.sync_copy(w_hbm, table_v)
        issue(0, den0, nom0, s0)

        def body(i, carry):
            ci = 2 * i
            wait_in(den0, nom0, s0)
            issue(ci + 1, den1, nom1, s1)

            @pl.when(i > 0)
            def _():
                wait_out(out0, so0)

            compute(den0, nom0, out0)
            pltpu.async_copy(out0, out_hbm.at[pl.ds(cbase2(ci), CP1 // 2)], so0)

            wait_in(den1, nom1, s1)
            issue(ci + 2, den0, nom0, s0)

            @pl.when(i > 0)
            def _():
                wait_out(out1, so1)

            compute(den1, nom1, out1)
            pltpu.async_copy(out1, out_hbm.at[pl.ds(cbase2(ci + 1), CP1 // 2)], so1)
            return carry

        lax.fori_loop(0, npairs, body, 0)
        wait_in(den0, nom0, s0)
        wait_out(out0, so0)
        wait_out(out1, so1)

        if PTAIL:
            @pl.when(wid == NW - 1)
            def _():
                for l in range(L):
                    for d in range(D):
                        pltpu.async_copy(
                            den_t.at[l, d, pl.ds(TB, PTAIL)],
                            den1.at[l, d, pl.ds(0, PTAIL)], s1,
                        )
                    pltpu.async_copy(
                        nom_t.at[l, pl.ds(TB, PTAIL)],
                        nom1.at[l, pl.ds(0, PTAIL)], s1,
                    )
                for l in range(L):
                    for d in range(D):
                        pltpu.make_async_copy(
                            den_t.at[l, d, pl.ds(TB, PTAIL)],
                            den1.at[l, d, pl.ds(0, PTAIL)], s1,
                        ).wait()
                    pltpu.make_async_copy(
                        nom_t.at[l, pl.ds(TB, PTAIL)],
                        nom1.at[l, pl.ds(0, PTAIL)], s1,
                    ).wait()
                for j in range(PTAIL // LANES):
                    block(den1, nom1, j)
                for j2 in range(PTAIL // (2 * LANES)):
                    pack2(out1, j2)
                pltpu.sync_copy(
                    out1.at[pl.ds(0, PTAIL // 2)],
                    out_hbm.at[pl.ds(TB // 2, PTAIL // 2)],
                )

    return k


def _make_stage2(V, K, P):
    NCH = V // CP2
    PTAIL = V % CP2
    TB = NCH * CP2
    PW = P // 2
    assert PTAIL % LANES == 0 and P % 32 == 0

    @functools.partial(
        pl.kernel,
        out_type=jax.ShapeDtypeStruct((V,), jnp.float32),
        mesh=_mesh(),
        compiler_params=_CPARAMS,
        scratch_types=[
            pltpu.VMEM((PW,), jnp.int32),
            pltpu.VMEM((K, CP2), jnp.int32),
            pltpu.VMEM((K, CP2), jnp.int32),
            pltpu.VMEM((CP2,), jnp.float32),
            pltpu.VMEM((CP2,), jnp.float32),
            pltpu.SemaphoreType.DMA,
            pltpu.SemaphoreType.DMA,
            pltpu.SemaphoreType.DMA,
            pltpu.SemaphoreType.DMA,
        ],
    )
    def k(v2p_t, pairs_hbm, out_hbm,
          pairs_v, vb0, vb1, out0, out1, s0, s1, so0, so1):
        wid = _wid()
        start_ch, cnt_ch = _chunk_range(wid, NCH)
        npairs = (cnt_ch + 1) // 2

        def cbase(ci):
            return (start_ch + jnp.minimum(ci, cnt_ch - 1)) * CP2

        def issue(ci, vb, sem):
            pltpu.async_copy(v2p_t.at[:, pl.ds(cbase(ci), CP2)], vb, sem)

        def wait_in(vb, sem):
            pltpu.make_async_copy(v2p_t.at[:, pl.ds(0, CP2)], vb, sem).wait()

        def wait_out(out_b, sem):
            pltpu.make_async_copy(out_b, out_hbm.at[pl.ds(0, CP2)], sem).wait()

        def block(vb, out_b, j):
            sl = pl.ds(j * LANES, LANES)
            acc = jnp.zeros((LANES,), jnp.float32)
            for kk in range(K):
                idx = vb[kk, sl]
                widx = ((idx >> 5) << 4) | (idx & 15)
                g = plsc.load_gather(pairs_v, [widx])
                hi = (idx & 16) != 0
                bits = jnp.where(hi, g & jnp.int32(-65536), g << 16)
                acc = acc + plsc.bitcast(bits, jnp.float32)
            out_b[sl] = acc

        pltpu.sync_copy(pairs_hbm, pairs_v)
        issue(0, vb0, s0)

        def body(i, carry):
            ci = 2 * i
            wait_in(vb0, s0)
            issue(ci + 1, vb1, s1)

            @pl.when(i > 0)
            def _():
                wait_out(out0, so0)

            @plsc.parallel_loop(0, CP2 // LANES, step=1, unroll=2)
            def _(j):
                block(vb0, out0, j)

            pltpu.async_copy(out0, out_hbm.at[pl.ds(cbase(ci), CP2)], so0)

            wait_in(vb1, s1)
            issue(ci + 2, vb0, s0)

            @pl.when(i > 0)
            def _():
                wait_out(out1, so1)

            @plsc.parallel_loop(0, CP2 // LANES, step=1, unroll=2)
            def _(j):
                block(vb1, out1, j)

            pltpu.async_copy(out1, out_hbm.at[pl.ds(cbase(ci + 1), CP2)], so1)
            return carry

        lax.fori_loop(0, npairs, body, 0)
        wait_in(vb0, s0)
        wait_out(out0, so0)
        wait_out(out1, so1)

        if PTAIL:
            @pl.when(wid == NW - 1)
            def _():
                for kk in range(K):
                    pltpu.async_copy(
                        v2p_t.at[kk, pl.ds(TB, PTAIL)],
                        vb1.at[kk, pl.ds(0, PTAIL)], s1,
                    )
                for kk in range(K):
                    pltpu.make_async_copy(
                        v2p_t.at[kk, pl.ds(TB, PTAIL)],
                        vb1.at[kk, pl.ds(0, PTAIL)], s1,
                    ).wait()
                for j in range(PTAIL // LANES):
                    block(vb1, out1, j)
                pltpu.sync_copy(
                    out1.at[pl.ds(0, PTAIL)], out_hbm.at[pl.ds(TB, PTAIL)]
                )

    return k


def kernel(variant_2_paths, paths_nom, paths_denom, w_transitions):
    V, K = variant_2_paths.shape
    P, L = paths_nom.shape
    D = paths_denom.shape[-1]
    T = w_transitions.shape[0]

    v2p_t = variant_2_paths.astype(jnp.int32).T
    nom_t = paths_nom.astype(jnp.int32).T
    den_t = jnp.transpose(paths_denom.astype(jnp.int32), (1, 2, 0))
    w = w_transitions.astype(jnp.float32)

    prob = _make_stage1(P, L, D, T)(nom_t, den_t, w)
    return _make_stage2(V, K, P)(v2p_t, prob)
